# baseline (speedup 1.0000x reference)
import jax
import jax.numpy as jnp
from jax import lax
from jax.experimental import pallas as pl
from jax.experimental.pallas import tpu as pltpu
from jax.experimental.pallas import tpu_sc as plsc

N = 10000
E = 320000
DIN = 128
DOUT = 128
R = 8
NB = 4

NPAD = 10240
NPLANES = R + 1
NC = 2
NS = 16
NW = NC * NS
EPW = 10240
EPAD = NW * EPW
CHUNK = 80
NCHUNK = EPW // CHUNK
NBUF = 2
ROWS_PER_TILE = NPAD // NS


_BLK = 1024


def _transform_body(wc_ref, wext_ref, x_ref, out_ref):
    r = pl.program_id(0)
    w = wc_ref[r, 0] * wext_ref[0]
    for b in range(1, NB + 1):
        w += wc_ref[r, b] * wext_ref[b]
    out_ref[0] = jnp.dot(x_ref[...], w, preferred_element_type=jnp.float32)


def _transform(xpad, w_ext, wc_ext):
    return pl.pallas_call(
        _transform_body,
        grid=(NPLANES, NPAD // _BLK),
        in_specs=[
            pl.BlockSpec(memory_space=pltpu.SMEM),
            pl.BlockSpec((NB + 1, DIN, DOUT), lambda r, j: (0, 0, 0)),
            pl.BlockSpec((_BLK, DIN), lambda r, j: (j, 0)),
        ],
        out_specs=pl.BlockSpec((1, _BLK, DOUT), lambda r, j: (r, j, 0)),
        out_shape=jax.ShapeDtypeStruct((NPLANES, NPAD, DOUT), jnp.float32),
    )(wc_ext, w_ext, xpad)


def _pack_body(t_ref, s_ref, d_ref, o_ref):
    g = t_ref[...] * NPAD + s_ref[...]
    blk = g.shape[0]
    o_ref[...] = jnp.concatenate(
        [g.reshape(blk, 1, CHUNK), d_ref[...].reshape(blk, 1, CHUNK)], axis=1)


def _pack_indices(t2, s2, d2):
    nch = EPAD // CHUNK
    blk = 256
    return pl.pallas_call(
        _pack_body,
        grid=(nch // blk,),
        in_specs=[
            pl.BlockSpec((blk, CHUNK), lambda i: (i, 0)),
            pl.BlockSpec((blk, CHUNK), lambda i: (i, 0)),
            pl.BlockSpec((blk, CHUNK), lambda i: (i, 0)),
        ],
        out_specs=pl.BlockSpec((blk, 2, CHUNK), lambda i: (i, 0, 0)),
        out_shape=jax.ShapeDtypeStruct((nch, 2, CHUNK), jnp.int32),
    )(t2, s2, d2)


def _sc_body(table_hbm, idx4_hbm, zeros_hbm,
             p0_hbm, p1_hbm,
             acc, idx_v, rows_v, *sems):
    c = lax.axis_index("c")
    s = lax.axis_index("s")
    wid = s * NC + c

    row0 = s * ROWS_PER_TILE

    @pl.when(c == 0)
    def _():
        pltpu.sync_copy(table_hbm.at[pl.ds(R * NPAD + row0, ROWS_PER_TILE)],
                        acc.at[pl.ds(row0, ROWS_PER_TILE)])

    @pl.when(c != 0)
    def _():
        pltpu.sync_copy(zeros_hbm, acc.at[pl.ds(row0, ROWS_PER_TILE)])

    plsc.subcore_barrier()

    isem = sems[:NBUF]
    gsem = sems[NBUF:]

    def start_idx(ci, b):
        pltpu.async_copy(idx4_hbm.at[wid, ci], idx_v.at[b], isem[b])

    def wait_idx(ci, b):
        pltpu.make_async_copy(idx4_hbm.at[wid, ci], idx_v.at[b],
                              isem[b]).wait()

    def start_gather(ci_b):
        pltpu.async_copy(table_hbm.at[idx_v.at[ci_b, 0]], rows_v.at[ci_b],
                         gsem[ci_b])

    def wait_gather(ci_b):
        pltpu.make_async_copy(table_hbm.at[idx_v.at[ci_b, 0]],
                              rows_v.at[ci_b], gsem[ci_b]).wait()

    for b in range(NBUF):
        start_idx(b, b)
    for b in range(NBUF - 1):
        wait_idx(b, b)
        start_gather(b)

    def phase(ci, b):
        bn = (b + NBUF - 1) % NBUF
        wait_idx(ci + NBUF - 1, bn)
        start_gather(bn)
        wait_gather(b)
        pltpu.sync_copy(rows_v.at[b], acc.at[idx_v.at[b, 1]], add=True)
        start_idx(ci + NBUF, b)

    def group_body(g, carry):
        for b in range(NBUF):
            phase(g * NBUF + b, b)
        return carry

    lax.fori_loop(0, NCHUNK // NBUF, group_body, 0)
    for b in range(NBUF - 1):
        wait_gather(b)
    wait_idx(NCHUNK + NBUF - 1, NBUF - 1)
    plsc.subcore_barrier()

    @pl.when(c == 0)
    def _():
        pltpu.sync_copy(acc.at[pl.ds(row0, ROWS_PER_TILE)],
                        p0_hbm.at[pl.ds(row0, ROWS_PER_TILE)])

    @pl.when(c != 0)
    def _():
        pltpu.sync_copy(acc.at[pl.ds(row0, ROWS_PER_TILE)],
                        p1_hbm.at[pl.ds(row0, ROWS_PER_TILE)])


def _sc_aggregate(table, idx4, zeros):
    mesh = plsc.VectorSubcoreMesh(core_axis_name="c", subcore_axis_name="s",
                                  num_cores=NC, num_subcores=NS)
    f = pl.kernel(
        _sc_body,
        out_type=[
            jax.ShapeDtypeStruct((NPAD, DOUT), jnp.float32),
            jax.ShapeDtypeStruct((NPAD, DOUT), jnp.float32),
        ],
        mesh=mesh,
        scratch_types=(
            [
                pltpu.VMEM_SHARED((NPAD, DOUT), jnp.float32),
                pltpu.VMEM((NBUF, 2, CHUNK), jnp.int32),
                pltpu.VMEM((NBUF, CHUNK, DOUT), jnp.float32),
            ]
            + [pltpu.SemaphoreType.DMA] * (2 * NBUF)
        ),
    )
    return f(table, idx4, zeros)


def _combine_body(a_ref, b_ref, o_ref):
    o_ref[...] = jnp.maximum(a_ref[...] + b_ref[...], 0.0)


def _combine(p0, p1):
    return pl.pallas_call(
        _combine_body,
        grid=(NPAD // _BLK,),
        in_specs=[
            pl.BlockSpec((_BLK, DOUT), lambda i: (i, 0)),
            pl.BlockSpec((_BLK, DOUT), lambda i: (i, 0)),
        ],
        out_specs=pl.BlockSpec((_BLK, DOUT), lambda i: (i, 0)),
        out_shape=jax.ShapeDtypeStruct((NPAD, DOUT), jnp.float32),
    )(p0, p1)


@jax.jit
def kernel(x, edge_index, edge_type, weight, w_comp, self_loop_weight):
    xpad = jnp.pad(x, ((0, NPAD - N), (0, 0)))
    w_ext = jnp.concatenate([weight, self_loop_weight[None]], axis=0)
    wc_ext = jnp.zeros((NPLANES, NB + 1), jnp.float32)
    wc_ext = wc_ext.at[:R, :NB].set(w_comp).at[R, NB].set(1.0)

    table3 = _transform(xpad, w_ext, wc_ext)
    table = table3.reshape(NPLANES * NPAD, DOUT)

    nch = EPAD // CHUNK
    srcp = jnp.pad(edge_index[0], (0, EPAD - E))
    typep = jnp.pad(edge_type, (0, EPAD - E))
    dstp = jnp.pad(edge_index[1], (0, EPAD - E),
                   constant_values=NPAD - 1)
    idx3 = _pack_indices(typep.reshape(nch, CHUNK),
                         srcp.reshape(nch, CHUNK),
                         dstp.reshape(nch, CHUNK))
    idx4 = idx3.reshape(NW, NCHUNK, 2, CHUNK)
    tail = jnp.zeros((NW, NBUF, 2, CHUNK), jnp.int32)
    tail = tail.at[:, :, 1, :].set(NPAD - 1)
    idx4 = jnp.concatenate([idx4, tail], axis=1)

    zeros = jnp.zeros((ROWS_PER_TILE, DOUT), jnp.float32)

    p0, p1 = _sc_aggregate(table, idx4, zeros)
    out = _combine(p0, p1)
    return out[:N]


# speedup vs baseline: 1.1840x; 1.1840x over previous
import jax
import jax.numpy as jnp
from jax import lax
from jax.experimental import pallas as pl
from jax.experimental.pallas import tpu as pltpu
from jax.experimental.pallas import tpu_sc as plsc

N = 10000
E = 320000
DIN = 128
DOUT = 128
R = 8
NB = 4

NPAD = 10240
NPLANES = R + 2
NC = 2
NS = 16
NW = NC * NS
EPW = 10240
EPAD = NW * EPW
CHUNK = 64
NCHUNK = EPW // CHUNK
NBUF = 2
ROWS_PER_TILE = NPAD // NS


_BLK = 1024


def _transform_body(wc_ref, wext_ref, x_ref, out_ref):
    r = pl.program_id(0)
    w = wc_ref[r, 0] * wext_ref[0]
    for b in range(1, NB + 1):
        w += wc_ref[r, b] * wext_ref[b]
    out_ref[0] = jnp.dot(x_ref[...], w, preferred_element_type=jnp.float32)


def _transform(xpad, w_ext, wc_ext):
    return pl.pallas_call(
        _transform_body,
        grid=(NPLANES, NPAD // _BLK),
        in_specs=[
            pl.BlockSpec(memory_space=pltpu.SMEM),
            pl.BlockSpec((NB + 1, DIN, DOUT), lambda r, j: (0, 0, 0)),
            pl.BlockSpec((_BLK, DIN), lambda r, j: (j, 0)),
        ],
        out_specs=pl.BlockSpec((1, _BLK, DOUT), lambda r, j: (r, j, 0)),
        out_shape=jax.ShapeDtypeStruct((NPLANES, NPAD, DOUT), jnp.float32),
    )(wc_ext, w_ext, xpad)


def _pack_body(t_ref, s_ref, d_ref, o_ref):
    g = t_ref[...] * NPAD + s_ref[...]
    blk = g.shape[0]
    o_ref[...] = jnp.concatenate(
        [g.reshape(blk, 1, CHUNK), d_ref[...].reshape(blk, 1, CHUNK)], axis=1)


def _pack_indices(t2, s2, d2):
    nch = EPAD // CHUNK
    blk = 256
    return pl.pallas_call(
        _pack_body,
        grid=(nch // blk,),
        in_specs=[
            pl.BlockSpec((blk, CHUNK), lambda i: (i, 0)),
            pl.BlockSpec((blk, CHUNK), lambda i: (i, 0)),
            pl.BlockSpec((blk, CHUNK), lambda i: (i, 0)),
        ],
        out_specs=pl.BlockSpec((blk, 2, CHUNK), lambda i: (i, 0, 0)),
        out_shape=jax.ShapeDtypeStruct((nch, 2, CHUNK), jnp.int32),
    )(t2, s2, d2)


def _sc_body(table_hbm, idx4_hbm, zeros_hbm,
             p0_hbm, p1_hbm,
             acc, idx_v, rows_v, *sems):
    c = lax.axis_index("c")
    s = lax.axis_index("s")
    wid = s * NC + c

    row0 = s * ROWS_PER_TILE
    pltpu.sync_copy(
        table_hbm.at[pl.ds((R + c) * NPAD + row0, ROWS_PER_TILE)],
        acc.at[pl.ds(row0, ROWS_PER_TILE)])

    plsc.subcore_barrier()

    isem = sems[:NBUF]
    gsem = sems[NBUF:]

    def start_idx(ci, b):
        pltpu.async_copy(idx4_hbm.at[wid, ci], idx_v.at[b], isem[b])

    def wait_idx(ci, b):
        pltpu.make_async_copy(idx4_hbm.at[wid, ci], idx_v.at[b],
                              isem[b]).wait()

    def start_gather(ci_b):
        pltpu.async_copy(table_hbm.at[idx_v.at[ci_b, 0]], rows_v.at[ci_b],
                         gsem[ci_b])

    def wait_gather(ci_b):
        pltpu.make_async_copy(table_hbm.at[idx_v.at[ci_b, 0]],
                              rows_v.at[ci_b], gsem[ci_b]).wait()

    for b in range(NBUF):
        start_idx(b, b)
    for b in range(NBUF - 1):
        wait_idx(b, b)
        start_gather(b)

    def phase(ci, b):
        bn = (b + NBUF - 1) % NBUF
        wait_idx(ci + NBUF - 1, bn)
        start_gather(bn)
        wait_gather(b)
        pltpu.sync_copy(rows_v.at[b], acc.at[idx_v.at[b, 1]], add=True)
        start_idx(ci + NBUF, b)

    def group_body(g, carry):
        for b in range(NBUF):
            phase(g * NBUF + b, b)
        return carry

    lax.fori_loop(0, NCHUNK // NBUF, group_body, 0)
    for b in range(NBUF - 1):
        wait_gather(b)
    wait_idx(NCHUNK + NBUF - 1, NBUF - 1)
    plsc.subcore_barrier()

    @pl.when(c == 0)
    def _():
        pltpu.sync_copy(acc.at[pl.ds(row0, ROWS_PER_TILE)],
                        p0_hbm.at[pl.ds(row0, ROWS_PER_TILE)])

    @pl.when(c != 0)
    def _():
        pltpu.sync_copy(acc.at[pl.ds(row0, ROWS_PER_TILE)],
                        p1_hbm.at[pl.ds(row0, ROWS_PER_TILE)])


def _sc_aggregate(table, idx4, zeros):
    mesh = plsc.VectorSubcoreMesh(core_axis_name="c", subcore_axis_name="s",
                                  num_cores=NC, num_subcores=NS)
    f = pl.kernel(
        _sc_body,
        out_type=[
            jax.ShapeDtypeStruct((NPAD, DOUT), jnp.float32),
            jax.ShapeDtypeStruct((NPAD, DOUT), jnp.float32),
        ],
        mesh=mesh,
        scratch_types=(
            [
                pltpu.VMEM_SHARED((NPAD, DOUT), jnp.float32),
                pltpu.VMEM((NBUF, 2, CHUNK), jnp.int32),
                pltpu.VMEM((NBUF, CHUNK, DOUT), jnp.float32),
            ]
            + [pltpu.SemaphoreType.DMA] * (2 * NBUF)
        ),
    )
    return f(table, idx4, zeros)


def _combine_body(a_ref, b_ref, o_ref):
    o_ref[...] = jnp.maximum(a_ref[...] + b_ref[...], 0.0)


def _combine(p0, p1):
    return pl.pallas_call(
        _combine_body,
        grid=(NPAD // _BLK,),
        in_specs=[
            pl.BlockSpec((_BLK, DOUT), lambda i: (i, 0)),
            pl.BlockSpec((_BLK, DOUT), lambda i: (i, 0)),
        ],
        out_specs=pl.BlockSpec((_BLK, DOUT), lambda i: (i, 0)),
        out_shape=jax.ShapeDtypeStruct((NPAD, DOUT), jnp.float32),
    )(p0, p1)


@jax.jit
def kernel(x, edge_index, edge_type, weight, w_comp, self_loop_weight):
    xpad = jnp.pad(x, ((0, NPAD - N), (0, 0)))
    w_ext = jnp.concatenate([weight, self_loop_weight[None]], axis=0)
    wc_ext = jnp.zeros((NPLANES, NB + 1), jnp.float32)
    wc_ext = wc_ext.at[:R, :NB].set(w_comp).at[R, NB].set(1.0)

    table3 = _transform(xpad, w_ext, wc_ext)
    table = table3.reshape(NPLANES * NPAD, DOUT)

    nch = EPAD // CHUNK
    srcp = jnp.pad(edge_index[0], (0, EPAD - E))
    typep = jnp.pad(edge_type, (0, EPAD - E))
    dstp = jnp.pad(edge_index[1], (0, EPAD - E),
                   constant_values=NPAD - 1)
    idx3 = _pack_indices(typep.reshape(nch, CHUNK),
                         srcp.reshape(nch, CHUNK),
                         dstp.reshape(nch, CHUNK))
    idx4 = idx3.reshape(NW, NCHUNK, 2, CHUNK)
    tail = jnp.zeros((NW, NBUF, 2, CHUNK), jnp.int32)
    tail = tail.at[:, :, 1, :].set(NPAD - 1)
    idx4 = jnp.concatenate([idx4, tail], axis=1)

    zeros = jnp.zeros((ROWS_PER_TILE, DOUT), jnp.float32)

    p0, p1 = _sc_aggregate(table, idx4, zeros)
    out = _combine(p0, p1)
    return out[:N]


# speedup vs baseline: 1.2704x; 1.0729x over previous
import jax
import jax.numpy as jnp
from jax import lax
from jax.experimental import pallas as pl
from jax.experimental.pallas import tpu as pltpu
from jax.experimental.pallas import tpu_sc as plsc

N = 10000
E = 320000
DIN = 128
DOUT = 128
R = 8
NB = 4

NPAD = 10240
NPLANES = R + 2
NC = 2
NS = 16
NW = NC * NS
EPW = 10240
EPAD = NW * EPW
CHUNK = 64
NCHUNK = EPW // CHUNK
NBUF = 2
NCHT = EPAD // CHUNK
CPT0 = 214
CPT1 = 106
ROWS_PER_TILE = NPAD // NS


_BLK = 1024


def _transform_body(wc_ref, wext_ref, x_ref, out_ref):
    r = pl.program_id(0)
    w = wc_ref[r, 0] * wext_ref[0]
    for b in range(1, NB + 1):
        w += wc_ref[r, b] * wext_ref[b]
    out_ref[0] = jnp.dot(x_ref[...], w, preferred_element_type=jnp.float32)


def _transform(xpad, w_ext, wc_ext):
    return pl.pallas_call(
        _transform_body,
        grid=(NPLANES, NPAD // _BLK),
        in_specs=[
            pl.BlockSpec(memory_space=pltpu.SMEM),
            pl.BlockSpec((NB + 1, DIN, DOUT), lambda r, j: (0, 0, 0)),
            pl.BlockSpec((_BLK, DIN), lambda r, j: (j, 0)),
        ],
        out_specs=pl.BlockSpec((1, _BLK, DOUT), lambda r, j: (r, j, 0)),
        out_shape=jax.ShapeDtypeStruct((NPLANES, NPAD, DOUT), jnp.float32),
    )(wc_ext, w_ext, xpad)


def _pack_body(t_ref, s_ref, d_ref, o_ref):
    g = t_ref[...] * NPAD + s_ref[...]
    blk = g.shape[0]
    o_ref[...] = jnp.concatenate(
        [g.reshape(blk, 1, CHUNK), d_ref[...].reshape(blk, 1, CHUNK)], axis=1)


def _pack_indices(t2, s2, d2):
    nch = EPAD // CHUNK
    blk = 256
    return pl.pallas_call(
        _pack_body,
        grid=(nch // blk,),
        in_specs=[
            pl.BlockSpec((blk, CHUNK), lambda i: (i, 0)),
            pl.BlockSpec((blk, CHUNK), lambda i: (i, 0)),
            pl.BlockSpec((blk, CHUNK), lambda i: (i, 0)),
        ],
        out_specs=pl.BlockSpec((blk, 2, CHUNK), lambda i: (i, 0, 0)),
        out_shape=jax.ShapeDtypeStruct((nch, 2, CHUNK), jnp.int32),
    )(t2, s2, d2)


def _sc_body(table_hbm, idx4_hbm, zeros_hbm,
             p0_hbm, p1_hbm,
             acc, idx_v, rows_v, *sems):
    c = lax.axis_index("c")
    s = lax.axis_index("s")

    row0 = s * ROWS_PER_TILE
    pltpu.sync_copy(
        table_hbm.at[pl.ds((R + c) * NPAD + row0, ROWS_PER_TILE)],
        acc.at[pl.ds(row0, ROWS_PER_TILE)])

    plsc.subcore_barrier()

    isem = sems[:NBUF]
    gsem = sems[NBUF:]

    def pipeline(count, base):
        def start_idx(ci, b):
            pltpu.async_copy(idx4_hbm.at[base + ci], idx_v.at[b], isem[b])

        def wait_idx(ci, b):
            pltpu.make_async_copy(idx4_hbm.at[base + ci], idx_v.at[b],
                                  isem[b]).wait()

        def start_gather(b):
            pltpu.async_copy(table_hbm.at[idx_v.at[b, 0]], rows_v.at[b],
                             gsem[b])

        def wait_gather(b):
            pltpu.make_async_copy(table_hbm.at[idx_v.at[b, 0]],
                                  rows_v.at[b], gsem[b]).wait()

        for b in range(NBUF):
            start_idx(b, b)
        for b in range(NBUF - 1):
            wait_idx(b, b)
            start_gather(b)

        def phase(ci, b):
            bn = (b + NBUF - 1) % NBUF
            wait_idx(ci + NBUF - 1, bn)
            start_gather(bn)
            wait_gather(b)
            pltpu.sync_copy(rows_v.at[b], acc.at[idx_v.at[b, 1]], add=True)
            start_idx(ci + NBUF, b)

        def group_body(g, carry):
            for b in range(NBUF):
                phase(g * NBUF + b, b)
            return carry

        lax.fori_loop(0, count // NBUF, group_body, 0)
        for b in range(NBUF - 1):
            wait_gather(b)
        wait_idx(count + NBUF - 1, NBUF - 1)

    @pl.when(c == 0)
    def _():
        pipeline(CPT0, s * CPT0)

    @pl.when(c != 0)
    def _():
        pipeline(CPT1, NS * CPT0 + s * CPT1)

    plsc.subcore_barrier()

    @pl.when(c == 0)
    def _():
        pltpu.sync_copy(acc.at[pl.ds(row0, ROWS_PER_TILE)],
                        p0_hbm.at[pl.ds(row0, ROWS_PER_TILE)])

    @pl.when(c != 0)
    def _():
        pltpu.sync_copy(acc.at[pl.ds(row0, ROWS_PER_TILE)],
                        p1_hbm.at[pl.ds(row0, ROWS_PER_TILE)])


def _sc_aggregate(table, idx4, zeros):
    mesh = plsc.VectorSubcoreMesh(core_axis_name="c", subcore_axis_name="s",
                                  num_cores=NC, num_subcores=NS)
    f = pl.kernel(
        _sc_body,
        out_type=[
            jax.ShapeDtypeStruct((NPAD, DOUT), jnp.float32),
            jax.ShapeDtypeStruct((NPAD, DOUT), jnp.float32),
        ],
        mesh=mesh,
        scratch_types=(
            [
                pltpu.VMEM_SHARED((NPAD, DOUT), jnp.float32),
                pltpu.VMEM((NBUF, 2, CHUNK), jnp.int32),
                pltpu.VMEM((NBUF, CHUNK, DOUT), jnp.float32),
            ]
            + [pltpu.SemaphoreType.DMA] * (2 * NBUF)
        ),
    )
    return f(table, idx4, zeros)


def _combine_body(a_ref, b_ref, o_ref):
    o_ref[...] = jnp.maximum(a_ref[...] + b_ref[...], 0.0)


def _combine(p0, p1):
    return pl.pallas_call(
        _combine_body,
        grid=(NPAD // _BLK,),
        in_specs=[
            pl.BlockSpec((_BLK, DOUT), lambda i: (i, 0)),
            pl.BlockSpec((_BLK, DOUT), lambda i: (i, 0)),
        ],
        out_specs=pl.BlockSpec((_BLK, DOUT), lambda i: (i, 0)),
        out_shape=jax.ShapeDtypeStruct((NPAD, DOUT), jnp.float32),
    )(p0, p1)


@jax.jit
def kernel(x, edge_index, edge_type, weight, w_comp, self_loop_weight):
    xpad = jnp.pad(x, ((0, NPAD - N), (0, 0)))
    w_ext = jnp.concatenate([weight, self_loop_weight[None]], axis=0)
    wc_ext = jnp.zeros((NPLANES, NB + 1), jnp.float32)
    wc_ext = wc_ext.at[:R, :NB].set(w_comp).at[R, NB].set(1.0)

    table3 = _transform(xpad, w_ext, wc_ext)
    table = table3.reshape(NPLANES * NPAD, DOUT)

    nch = EPAD // CHUNK
    srcp = jnp.pad(edge_index[0], (0, EPAD - E))
    typep = jnp.pad(edge_type, (0, EPAD - E))
    dstp = jnp.pad(edge_index[1], (0, EPAD - E),
                   constant_values=NPAD - 1)
    idx3 = _pack_indices(typep.reshape(nch, CHUNK),
                         srcp.reshape(nch, CHUNK),
                         dstp.reshape(nch, CHUNK))
    tail = jnp.zeros((NBUF, 2, CHUNK), jnp.int32)
    idx4 = jnp.concatenate([idx3, tail], axis=0)

    zeros = jnp.zeros((ROWS_PER_TILE, DOUT), jnp.float32)

    p0, p1 = _sc_aggregate(table, idx4, zeros)
    out = _combine(p0, p1)
    return out[:N]


# speedup vs baseline: 1.2760x; 1.0044x over previous
import jax
import jax.numpy as jnp
from jax import lax
from jax.experimental import pallas as pl
from jax.experimental.pallas import tpu as pltpu
from jax.experimental.pallas import tpu_sc as plsc

N = 10000
E = 320000
DIN = 128
DOUT = 128
R = 8
NB = 4

NPAD = 10240
NPLANES = R + 2
NC = 2
NS = 16
NW = NC * NS
EPW = 10240
EPAD = NW * EPW
CHUNK = 64
NCHUNK = EPW // CHUNK
NBUF = 2
NCHT = EPAD // CHUNK
CPT0 = 240
CPT1 = 80
ROWS_PER_TILE = NPAD // NS


_BLK = 1024


def _transform_body(wc_ref, wext_ref, x_ref, out_ref):
    r = pl.program_id(0)
    w = wc_ref[r, 0] * wext_ref[0]
    for b in range(1, NB + 1):
        w += wc_ref[r, b] * wext_ref[b]
    out_ref[0] = jnp.dot(x_ref[...], w, preferred_element_type=jnp.float32)


def _transform(xpad, w_ext, wc_ext):
    return pl.pallas_call(
        _transform_body,
        grid=(NPLANES, NPAD // _BLK),
        in_specs=[
            pl.BlockSpec(memory_space=pltpu.SMEM),
            pl.BlockSpec((NB + 1, DIN, DOUT), lambda r, j: (0, 0, 0)),
            pl.BlockSpec((_BLK, DIN), lambda r, j: (j, 0)),
        ],
        out_specs=pl.BlockSpec((1, _BLK, DOUT), lambda r, j: (r, j, 0)),
        out_shape=jax.ShapeDtypeStruct((NPLANES, NPAD, DOUT), jnp.float32),
    )(wc_ext, w_ext, xpad)


def _pack_body(t_ref, s_ref, d_ref, o_ref):
    g = t_ref[...] * NPAD + s_ref[...]
    blk = g.shape[0]
    o_ref[...] = jnp.concatenate(
        [g.reshape(blk, 1, CHUNK), d_ref[...].reshape(blk, 1, CHUNK)], axis=1)


def _pack_indices(t2, s2, d2):
    nch = EPAD // CHUNK
    blk = 256
    return pl.pallas_call(
        _pack_body,
        grid=(nch // blk,),
        in_specs=[
            pl.BlockSpec((blk, CHUNK), lambda i: (i, 0)),
            pl.BlockSpec((blk, CHUNK), lambda i: (i, 0)),
            pl.BlockSpec((blk, CHUNK), lambda i: (i, 0)),
        ],
        out_specs=pl.BlockSpec((blk, 2, CHUNK), lambda i: (i, 0, 0)),
        out_shape=jax.ShapeDtypeStruct((nch, 2, CHUNK), jnp.int32),
    )(t2, s2, d2)


def _sc_body(table_hbm, idx4_hbm, zeros_hbm,
             p0_hbm, p1_hbm,
             acc, idx_v, rows_v, *sems):
    c = lax.axis_index("c")
    s = lax.axis_index("s")

    row0 = s * ROWS_PER_TILE
    pltpu.sync_copy(
        table_hbm.at[pl.ds((R + c) * NPAD + row0, ROWS_PER_TILE)],
        acc.at[pl.ds(row0, ROWS_PER_TILE)])

    plsc.subcore_barrier()

    isem = sems[:NBUF]
    gsem = sems[NBUF:]

    def pipeline(count, base):
        def start_idx(ci, b):
            pltpu.async_copy(idx4_hbm.at[base + ci], idx_v.at[b], isem[b])

        def wait_idx(ci, b):
            pltpu.make_async_copy(idx4_hbm.at[base + ci], idx_v.at[b],
                                  isem[b]).wait()

        def start_gather(b):
            pltpu.async_copy(table_hbm.at[idx_v.at[b, 0]], rows_v.at[b],
                             gsem[b])

        def wait_gather(b):
            pltpu.make_async_copy(table_hbm.at[idx_v.at[b, 0]],
                                  rows_v.at[b], gsem[b]).wait()

        for b in range(NBUF):
            start_idx(b, b)
        for b in range(NBUF - 1):
            wait_idx(b, b)
            start_gather(b)

        def phase(ci, b):
            bn = (b + NBUF - 1) % NBUF
            wait_idx(ci + NBUF - 1, bn)
            start_gather(bn)
            wait_gather(b)
            pltpu.sync_copy(rows_v.at[b], acc.at[idx_v.at[b, 1]], add=True)
            start_idx(ci + NBUF, b)

        def group_body(g, carry):
            for b in range(NBUF):
                phase(g * NBUF + b, b)
            return carry

        lax.fori_loop(0, count // NBUF, group_body, 0)
        for b in range(NBUF - 1):
            wait_gather(b)
        wait_idx(count + NBUF - 1, NBUF - 1)

    @pl.when(c == 0)
    def _():
        pipeline(CPT0, s * CPT0)

    @pl.when(c != 0)
    def _():
        pipeline(CPT1, NS * CPT0 + s * CPT1)

    plsc.subcore_barrier()

    @pl.when(c == 0)
    def _():
        pltpu.sync_copy(acc.at[pl.ds(row0, ROWS_PER_TILE)],
                        p0_hbm.at[pl.ds(row0, ROWS_PER_TILE)])

    @pl.when(c != 0)
    def _():
        pltpu.sync_copy(acc.at[pl.ds(row0, ROWS_PER_TILE)],
                        p1_hbm.at[pl.ds(row0, ROWS_PER_TILE)])


def _sc_aggregate(table, idx4, zeros):
    mesh = plsc.VectorSubcoreMesh(core_axis_name="c", subcore_axis_name="s",
                                  num_cores=NC, num_subcores=NS)
    f = pl.kernel(
        _sc_body,
        out_type=[
            jax.ShapeDtypeStruct((NPAD, DOUT), jnp.float32),
            jax.ShapeDtypeStruct((NPAD, DOUT), jnp.float32),
        ],
        mesh=mesh,
        scratch_types=(
            [
                pltpu.VMEM_SHARED((NPAD, DOUT), jnp.float32),
                pltpu.VMEM((NBUF, 2, CHUNK), jnp.int32),
                pltpu.VMEM((NBUF, CHUNK, DOUT), jnp.float32),
            ]
            + [pltpu.SemaphoreType.DMA] * (2 * NBUF)
        ),
    )
    return f(table, idx4, zeros)


def _combine_body(a_ref, b_ref, o_ref):
    o_ref[...] = jnp.maximum(a_ref[...] + b_ref[...], 0.0)


def _combine(p0, p1):
    return pl.pallas_call(
        _combine_body,
        grid=(NPAD // _BLK,),
        in_specs=[
            pl.BlockSpec((_BLK, DOUT), lambda i: (i, 0)),
            pl.BlockSpec((_BLK, DOUT), lambda i: (i, 0)),
        ],
        out_specs=pl.BlockSpec((_BLK, DOUT), lambda i: (i, 0)),
        out_shape=jax.ShapeDtypeStruct((NPAD, DOUT), jnp.float32),
    )(p0, p1)


@jax.jit
def kernel(x, edge_index, edge_type, weight, w_comp, self_loop_weight):
    xpad = jnp.pad(x, ((0, NPAD - N), (0, 0)))
    w_ext = jnp.concatenate([weight, self_loop_weight[None]], axis=0)
    wc_ext = jnp.zeros((NPLANES, NB + 1), jnp.float32)
    wc_ext = wc_ext.at[:R, :NB].set(w_comp).at[R, NB].set(1.0)

    table3 = _transform(xpad, w_ext, wc_ext)
    table = table3.reshape(NPLANES * NPAD, DOUT)

    nch = EPAD // CHUNK
    srcp = jnp.pad(edge_index[0], (0, EPAD - E))
    typep = jnp.pad(edge_type, (0, EPAD - E))
    dstp = jnp.pad(edge_index[1], (0, EPAD - E),
                   constant_values=NPAD - 1)
    idx3 = _pack_indices(typep.reshape(nch, CHUNK),
                         srcp.reshape(nch, CHUNK),
                         dstp.reshape(nch, CHUNK))
    tail = jnp.zeros((NBUF, 2, CHUNK), jnp.int32)
    idx4 = jnp.concatenate([idx3, tail], axis=0)

    zeros = jnp.zeros((ROWS_PER_TILE, DOUT), jnp.float32)

    p0, p1 = _sc_aggregate(table, idx4, zeros)
    out = _combine(p0, p1)
    return out[:N]


# speedup vs baseline: 1.2820x; 1.0048x over previous
import jax
import jax.numpy as jnp
from jax import lax
from jax.experimental import pallas as pl
from jax.experimental.pallas import tpu as pltpu
from jax.experimental.pallas import tpu_sc as plsc

N = 10000
E = 320000
DIN = 128
DOUT = 128
R = 8
NB = 4

NPAD = 10240
NPLANES = R + 2
NC = 2
NS = 16
NW = NC * NS
EPW = 10240
EPAD = NW * EPW
CHUNK = 64
NCHUNK = EPW // CHUNK
NBUF = 2
NCHT = EPAD // CHUNK
CPT0 = 256
CPT1 = 64
ROWS_PER_TILE = NPAD // NS


_BLK = 1024


def _transform_body(wc_ref, wext_ref, x_ref, out_ref):
    r = pl.program_id(0)
    w = wc_ref[r, 0] * wext_ref[0]
    for b in range(1, NB + 1):
        w += wc_ref[r, b] * wext_ref[b]
    out_ref[0] = jnp.dot(x_ref[...], w, preferred_element_type=jnp.float32)


def _transform(xpad, w_ext, wc_ext):
    return pl.pallas_call(
        _transform_body,
        grid=(NPLANES, NPAD // _BLK),
        in_specs=[
            pl.BlockSpec(memory_space=pltpu.SMEM),
            pl.BlockSpec((NB + 1, DIN, DOUT), lambda r, j: (0, 0, 0)),
            pl.BlockSpec((_BLK, DIN), lambda r, j: (j, 0)),
        ],
        out_specs=pl.BlockSpec((1, _BLK, DOUT), lambda r, j: (r, j, 0)),
        out_shape=jax.ShapeDtypeStruct((NPLANES, NPAD, DOUT), jnp.float32),
    )(wc_ext, w_ext, xpad)


def _pack_body(t_ref, s_ref, d_ref, o_ref):
    g = t_ref[...] * NPAD + s_ref[...]
    blk = g.shape[0]
    o_ref[...] = jnp.concatenate(
        [g.reshape(blk, 1, CHUNK), d_ref[...].reshape(blk, 1, CHUNK)], axis=1)


def _pack_indices(t2, s2, d2):
    nch = EPAD // CHUNK
    blk = 256
    return pl.pallas_call(
        _pack_body,
        grid=(nch // blk,),
        in_specs=[
            pl.BlockSpec((blk, CHUNK), lambda i: (i, 0)),
            pl.BlockSpec((blk, CHUNK), lambda i: (i, 0)),
            pl.BlockSpec((blk, CHUNK), lambda i: (i, 0)),
        ],
        out_specs=pl.BlockSpec((blk, 2, CHUNK), lambda i: (i, 0, 0)),
        out_shape=jax.ShapeDtypeStruct((nch, 2, CHUNK), jnp.int32),
    )(t2, s2, d2)


def _sc_body(table_hbm, idx4_hbm, zeros_hbm,
             p0_hbm, p1_hbm,
             acc, idx_v, rows_v, *sems):
    c = lax.axis_index("c")
    s = lax.axis_index("s")

    row0 = s * ROWS_PER_TILE
    pltpu.sync_copy(
        table_hbm.at[pl.ds((R + c) * NPAD + row0, ROWS_PER_TILE)],
        acc.at[pl.ds(row0, ROWS_PER_TILE)])

    plsc.subcore_barrier()

    isem = sems[:NBUF]
    gsem = sems[NBUF:]

    def pipeline(count, base):
        def start_idx(ci, b):
            pltpu.async_copy(idx4_hbm.at[base + ci], idx_v.at[b], isem[b])

        def wait_idx(ci, b):
            pltpu.make_async_copy(idx4_hbm.at[base + ci], idx_v.at[b],
                                  isem[b]).wait()

        def start_gather(b):
            pltpu.async_copy(table_hbm.at[idx_v.at[b, 0]], rows_v.at[b],
                             gsem[b])

        def wait_gather(b):
            pltpu.make_async_copy(table_hbm.at[idx_v.at[b, 0]],
                                  rows_v.at[b], gsem[b]).wait()

        for b in range(NBUF):
            start_idx(b, b)
        for b in range(NBUF - 1):
            wait_idx(b, b)
            start_gather(b)

        def phase(ci, b):
            bn = (b + NBUF - 1) % NBUF
            wait_idx(ci + NBUF - 1, bn)
            start_gather(bn)
            wait_gather(b)
            pltpu.sync_copy(rows_v.at[b], acc.at[idx_v.at[b, 1]], add=True)
            start_idx(ci + NBUF, b)

        def group_body(g, carry):
            for b in range(NBUF):
                phase(g * NBUF + b, b)
            return carry

        lax.fori_loop(0, count // NBUF, group_body, 0)
        for b in range(NBUF - 1):
            wait_gather(b)
        wait_idx(count + NBUF - 1, NBUF - 1)

    @pl.when(c == 0)
    def _():
        pipeline(CPT0, s * CPT0)

    @pl.when(c != 0)
    def _():
        pipeline(CPT1, NS * CPT0 + s * CPT1)

    plsc.subcore_barrier()

    @pl.when(c == 0)
    def _():
        pltpu.sync_copy(acc.at[pl.ds(row0, ROWS_PER_TILE)],
                        p0_hbm.at[pl.ds(row0, ROWS_PER_TILE)])

    @pl.when(c != 0)
    def _():
        pltpu.sync_copy(acc.at[pl.ds(row0, ROWS_PER_TILE)],
                        p1_hbm.at[pl.ds(row0, ROWS_PER_TILE)])


def _sc_aggregate(table, idx4, zeros):
    mesh = plsc.VectorSubcoreMesh(core_axis_name="c", subcore_axis_name="s",
                                  num_cores=NC, num_subcores=NS)
    f = pl.kernel(
        _sc_body,
        out_type=[
            jax.ShapeDtypeStruct((NPAD, DOUT), jnp.float32),
            jax.ShapeDtypeStruct((NPAD, DOUT), jnp.float32),
        ],
        mesh=mesh,
        scratch_types=(
            [
                pltpu.VMEM_SHARED((NPAD, DOUT), jnp.float32),
                pltpu.VMEM((NBUF, 2, CHUNK), jnp.int32),
                pltpu.VMEM((NBUF, CHUNK, DOUT), jnp.float32),
            ]
            + [pltpu.SemaphoreType.DMA] * (2 * NBUF)
        ),
    )
    return f(table, idx4, zeros)


def _combine_body(a_ref, b_ref, o_ref):
    o_ref[...] = jnp.maximum(a_ref[...] + b_ref[...], 0.0)


def _combine(p0, p1):
    return pl.pallas_call(
        _combine_body,
        grid=(NPAD // _BLK,),
        in_specs=[
            pl.BlockSpec((_BLK, DOUT), lambda i: (i, 0)),
            pl.BlockSpec((_BLK, DOUT), lambda i: (i, 0)),
        ],
        out_specs=pl.BlockSpec((_BLK, DOUT), lambda i: (i, 0)),
        out_shape=jax.ShapeDtypeStruct((NPAD, DOUT), jnp.float32),
    )(p0, p1)


@jax.jit
def kernel(x, edge_index, edge_type, weight, w_comp, self_loop_weight):
    xpad = jnp.pad(x, ((0, NPAD - N), (0, 0)))
    w_ext = jnp.concatenate([weight, self_loop_weight[None]], axis=0)
    wc_ext = jnp.zeros((NPLANES, NB + 1), jnp.float32)
    wc_ext = wc_ext.at[:R, :NB].set(w_comp).at[R, NB].set(1.0)

    table3 = _transform(xpad, w_ext, wc_ext)
    table = table3.reshape(NPLANES * NPAD, DOUT)

    nch = EPAD // CHUNK
    srcp = jnp.pad(edge_index[0], (0, EPAD - E))
    typep = jnp.pad(edge_type, (0, EPAD - E))
    dstp = jnp.pad(edge_index[1], (0, EPAD - E),
                   constant_values=NPAD - 1)
    idx3 = _pack_indices(typep.reshape(nch, CHUNK),
                         srcp.reshape(nch, CHUNK),
                         dstp.reshape(nch, CHUNK))
    tail = jnp.zeros((NBUF, 2, CHUNK), jnp.int32)
    idx4 = jnp.concatenate([idx3, tail], axis=0)

    zeros = jnp.zeros((ROWS_PER_TILE, DOUT), jnp.float32)

    p0, p1 = _sc_aggregate(table, idx4, zeros)
    out = _combine(p0, p1)
    return out[:N]


# speedup vs baseline: 1.4431x; 1.1256x over previous
import jax
import jax.numpy as jnp
from jax import lax
from jax.experimental import pallas as pl
from jax.experimental.pallas import tpu as pltpu
from jax.experimental.pallas import tpu_sc as plsc

N = 10000
E = 320000
DIN = 128
DOUT = 128
R = 8
NB = 4

NPAD = 10240
NPLANES = R + 2
NC = 2
NS = 16
NW = NC * NS
EPW = 10240
EPAD = NW * EPW
CHUNK = 64
NCHUNK = EPW // CHUNK
NBUF = 2
NCHT = EPAD // CHUNK
CPT0 = 288
CPT1 = 32
ROWS_PER_TILE = NPAD // NS


_BLK = 1024


def _transform_body(wc_ref, wext_ref, x_ref, out_ref):
    r = pl.program_id(0)
    w = wc_ref[r, 0] * wext_ref[0]
    for b in range(1, NB + 1):
        w += wc_ref[r, b] * wext_ref[b]
    out_ref[0] = jnp.dot(x_ref[...], w, preferred_element_type=jnp.float32)


def _transform(xpad, w_ext, wc_ext):
    return pl.pallas_call(
        _transform_body,
        grid=(NPLANES, NPAD // _BLK),
        in_specs=[
            pl.BlockSpec(memory_space=pltpu.SMEM),
            pl.BlockSpec((NB + 1, DIN, DOUT), lambda r, j: (0, 0, 0)),
            pl.BlockSpec((_BLK, DIN), lambda r, j: (j, 0)),
        ],
        out_specs=pl.BlockSpec((1, _BLK, DOUT), lambda r, j: (r, j, 0)),
        out_shape=jax.ShapeDtypeStruct((NPLANES, NPAD, DOUT), jnp.float32),
    )(wc_ext, w_ext, xpad)


def _pack_body(t_ref, s_ref, d_ref, o_ref):
    g = t_ref[...] * NPAD + s_ref[...]
    blk = g.shape[0]
    o_ref[...] = jnp.concatenate(
        [g.reshape(blk, 1, CHUNK), d_ref[...].reshape(blk, 1, CHUNK)], axis=1)


def _pack_indices(t2, s2, d2):
    nch = EPAD // CHUNK
    blk = 256
    return pl.pallas_call(
        _pack_body,
        grid=(nch // blk,),
        in_specs=[
            pl.BlockSpec((blk, CHUNK), lambda i: (i, 0)),
            pl.BlockSpec((blk, CHUNK), lambda i: (i, 0)),
            pl.BlockSpec((blk, CHUNK), lambda i: (i, 0)),
        ],
        out_specs=pl.BlockSpec((blk, 2, CHUNK), lambda i: (i, 0, 0)),
        out_shape=jax.ShapeDtypeStruct((nch, 2, CHUNK), jnp.int32),
    )(t2, s2, d2)


def _sc_body(table_hbm, idx4_hbm, zeros_hbm,
             p0_hbm, p1_hbm,
             acc, idx_v, rows_v, *sems):
    c = lax.axis_index("c")
    s = lax.axis_index("s")

    row0 = s * ROWS_PER_TILE
    pltpu.sync_copy(
        table_hbm.at[pl.ds((R + c) * NPAD + row0, ROWS_PER_TILE)],
        acc.at[pl.ds(row0, ROWS_PER_TILE)])

    plsc.subcore_barrier()

    isem = sems[:NBUF]
    gsem = sems[NBUF:]

    def pipeline(count, base):
        def start_idx(ci, b):
            pltpu.async_copy(idx4_hbm.at[base + ci], idx_v.at[b], isem[b])

        def wait_idx(ci, b):
            pltpu.make_async_copy(idx4_hbm.at[base + ci], idx_v.at[b],
                                  isem[b]).wait()

        def start_gather(b):
            pltpu.async_copy(table_hbm.at[idx_v.at[b, 0]], rows_v.at[b],
                             gsem[b])

        def wait_gather(b):
            pltpu.make_async_copy(table_hbm.at[idx_v.at[b, 0]],
                                  rows_v.at[b], gsem[b]).wait()

        for b in range(NBUF):
            start_idx(b, b)
        for b in range(NBUF - 1):
            wait_idx(b, b)
            start_gather(b)

        def phase(ci, b):
            bn = (b + NBUF - 1) % NBUF
            wait_idx(ci + NBUF - 1, bn)
            start_gather(bn)
            wait_gather(b)
            pltpu.sync_copy(rows_v.at[b], acc.at[idx_v.at[b, 1]], add=True)
            start_idx(ci + NBUF, b)

        def group_body(g, carry):
            for b in range(NBUF):
                phase(g * NBUF + b, b)
            return carry

        lax.fori_loop(0, count // NBUF, group_body, 0)
        for b in range(NBUF - 1):
            wait_gather(b)
        wait_idx(count + NBUF - 1, NBUF - 1)

    @pl.when(c == 0)
    def _():
        pipeline(CPT0, s * CPT0)

    @pl.when(c != 0)
    def _():
        pipeline(CPT1, NS * CPT0 + s * CPT1)

    plsc.subcore_barrier()

    @pl.when(c == 0)
    def _():
        pltpu.sync_copy(acc.at[pl.ds(row0, ROWS_PER_TILE)],
                        p0_hbm.at[pl.ds(row0, ROWS_PER_TILE)])

    @pl.when(c != 0)
    def _():
        pltpu.sync_copy(acc.at[pl.ds(row0, ROWS_PER_TILE)],
                        p1_hbm.at[pl.ds(row0, ROWS_PER_TILE)])


def _sc_aggregate(table, idx4, zeros):
    mesh = plsc.VectorSubcoreMesh(core_axis_name="c", subcore_axis_name="s",
                                  num_cores=NC, num_subcores=NS)
    f = pl.kernel(
        _sc_body,
        out_type=[
            jax.ShapeDtypeStruct((NPAD, DOUT), jnp.float32),
            jax.ShapeDtypeStruct((NPAD, DOUT), jnp.float32),
        ],
        mesh=mesh,
        scratch_types=(
            [
                pltpu.VMEM_SHARED((NPAD, DOUT), jnp.float32),
                pltpu.VMEM((NBUF, 2, CHUNK), jnp.int32),
                pltpu.VMEM((NBUF, CHUNK, DOUT), jnp.float32),
            ]
            + [pltpu.SemaphoreType.DMA] * (2 * NBUF)
        ),
    )
    return f(table, idx4, zeros)


def _combine_body(a_ref, b_ref, o_ref):
    o_ref[...] = jnp.maximum(a_ref[...] + b_ref[...], 0.0)


def _combine(p0, p1):
    return pl.pallas_call(
        _combine_body,
        grid=(NPAD // _BLK,),
        in_specs=[
            pl.BlockSpec((_BLK, DOUT), lambda i: (i, 0)),
            pl.BlockSpec((_BLK, DOUT), lambda i: (i, 0)),
        ],
        out_specs=pl.BlockSpec((_BLK, DOUT), lambda i: (i, 0)),
        out_shape=jax.ShapeDtypeStruct((NPAD, DOUT), jnp.float32),
    )(p0, p1)


@jax.jit
def kernel(x, edge_index, edge_type, weight, w_comp, self_loop_weight):
    xpad = jnp.pad(x, ((0, NPAD - N), (0, 0)))
    w_ext = jnp.concatenate([weight, self_loop_weight[None]], axis=0)
    wc_ext = jnp.zeros((NPLANES, NB + 1), jnp.float32)
    wc_ext = wc_ext.at[:R, :NB].set(w_comp).at[R, NB].set(1.0)

    table3 = _transform(xpad, w_ext, wc_ext)
    table = table3.reshape(NPLANES * NPAD, DOUT)

    nch = EPAD // CHUNK
    srcp = jnp.pad(edge_index[0], (0, EPAD - E))
    typep = jnp.pad(edge_type, (0, EPAD - E))
    dstp = jnp.pad(edge_index[1], (0, EPAD - E),
                   constant_values=NPAD - 1)
    idx3 = _pack_indices(typep.reshape(nch, CHUNK),
                         srcp.reshape(nch, CHUNK),
                         dstp.reshape(nch, CHUNK))
    tail = jnp.zeros((NBUF, 2, CHUNK), jnp.int32)
    idx4 = jnp.concatenate([idx3, tail], axis=0)

    zeros = jnp.zeros((ROWS_PER_TILE, DOUT), jnp.float32)

    p0, p1 = _sc_aggregate(table, idx4, zeros)
    out = _combine(p0, p1)
    return out[:N]
